# Initial kernel scaffold; baseline (speedup 1.0000x reference)
import jax
import jax.numpy as jnp
from jax.experimental import pallas as pl


def kernel(sources, destinations, timestamps, edge_features, memory, last_update, W1, b1, W2, b2, W_ih, W_hh, b_ih, b_hh):
    raise NotImplementedError("write your pallas kernel here")


# speedup vs baseline: 3.7773x; 3.7773x over previous
import jax
import jax.numpy as jnp
from jax import lax
from jax.experimental import pallas as pl
from jax.experimental.pallas import tpu as pltpu
from jax.experimental.pallas import tpu_sc as plsc

N = 10000
E = 320000
MEM = 128
EDGE_F = 16
RAW = 2 * MEM + 1 + EDGE_F
HID = RAW // 2
D = 144

NC = 2
NS = 16
NW = NC * NS
EPT = E // NW
K = 80
NCH = EPT // K
NP = 10000
ZR = NP // NS
CW = 8


def _node_proj_body(mem_ref, lu_ref, w1a_ref, w1b_ref, wt_ref, g1_ref, g2_ref):
    m = mem_ref[...]
    g1_ref[...] = (jnp.dot(m, w1a_ref[...], preferred_element_type=jnp.float32)
                   - lu_ref[...] * wt_ref[...])
    g2_ref[...] = jnp.dot(m, w1b_ref[...], preferred_element_type=jnp.float32)


def _edge_const_body(ef_ref, ts_ref, w1d_ref, wt_ref, b1_ref, c_ref):
    c_ref[...] = (jnp.dot(ef_ref[...], w1d_ref[...], preferred_element_type=jnp.float32)
                  + ts_ref[...] * wt_ref[...] + b1_ref[...])


def _sc_body(g1_hbm, g2_hbm, c_hbm, src_hbm, dst_hbm, zrow_hbm, z16_hbm, ones_hbm,
             h_out, cnt_out,
             src_v, dst_v, g1r, g2r, cr, ones_v, h_sh, c_sh, sem1, sem2):
    cid = lax.axis_index("c")
    sid = lax.axis_index("s")
    pltpu.sync_copy(zrow_hbm, h_sh.at[pl.ds(sid * ZR, ZR)])
    pltpu.sync_copy(z16_hbm, c_sh.at[pl.ds(sid * ZR, ZR)])
    pltpu.sync_copy(ones_hbm, ones_v)
    plsc.subcore_barrier()

    base = (cid * NS + sid) * EPT

    @pl.loop(0, NCH)
    def _chunk(ch):
        off = pl.multiple_of(base + ch * K, 8)
        pltpu.sync_copy(src_hbm.at[pl.ds(off, K)], src_v)
        pltpu.sync_copy(dst_hbm.at[pl.ds(off, K)], dst_v)
        cp1 = pltpu.async_copy(g1_hbm.at[src_v], g1r, sem1)
        cp2 = pltpu.async_copy(g2_hbm.at[dst_v], g2r, sem2)
        pltpu.sync_copy(c_hbm.at[pl.ds(off, K)], cr)
        cp1.wait()
        cp2.wait()

        @pl.loop(0, K)
        def _row(r):
            for j in range(D // 16):
                s = pl.ds(j * 16, 16)
                v = g1r[r, s] + g2r[r, s] + cr[r, s]
                g1r[r, s] = jnp.maximum(v, 0.0)

        pltpu.sync_copy(g1r, h_sh.at[src_v], add=True)
        pltpu.sync_copy(ones_v, c_sh.at[src_v], add=True)

    plsc.subcore_barrier()
    pltpu.sync_copy(h_sh.at[pl.ds(sid * ZR, ZR)], h_out.at[cid, pl.ds(sid * ZR, ZR)])
    pltpu.sync_copy(c_sh.at[pl.ds(sid * ZR, ZR)], cnt_out.at[cid, pl.ds(sid * ZR, ZR)])


def _finish_body(hs_ref, cs_ref, mem_ref, w2_ref, b2_ref, wih_ref, whh_ref,
                 bih_ref, bhh_ref, out_ref):
    hsum = hs_ref[0] + hs_ref[1]
    cnt = cs_ref[0, :, 0:1] + cs_ref[1, :, 0:1]
    mean = hsum / jnp.maximum(cnt, 1.0)
    agg = jnp.dot(mean, w2_ref[...], preferred_element_type=jnp.float32) + b2_ref[...]
    m = mem_ref[...]
    gi = jnp.dot(agg, wih_ref[...], preferred_element_type=jnp.float32) + bih_ref[...]
    gh = jnp.dot(m, whh_ref[...], preferred_element_type=jnp.float32) + bhh_ref[...]
    r = jax.nn.sigmoid(gi[:, :MEM] + gh[:, :MEM])
    z = jax.nn.sigmoid(gi[:, MEM:2 * MEM] + gh[:, MEM:2 * MEM])
    n = jnp.tanh(gi[:, 2 * MEM:] + r * gh[:, 2 * MEM:])
    new = (1.0 - z) * n + z * m
    out_ref[...] = jnp.where(cnt > 0.0, new, m)


def kernel(sources, destinations, timestamps, edge_features, memory, last_update,
           W1, b1, W2, b2, W_ih, W_hh, b_ih, b_hh):
    f32 = jnp.float32
    pad = lambda w: jnp.pad(w, ((0, 0), (0, D - HID)))
    w1a = pad(W1[:MEM])
    w1b = pad(W1[MEM:2 * MEM])
    wt = pad(W1[2 * MEM:2 * MEM + 1])
    w1d = pad(W1[2 * MEM + 1:])
    b1p = pad(b1[None, :])
    w2p = jnp.pad(W2, ((0, D - HID), (0, 0)))
    lu = last_update[:, None]
    ts = timestamps[:, None]

    BN = 2000
    g1, g2 = pl.pallas_call(
        _node_proj_body,
        grid=(N // BN,),
        in_specs=[
            pl.BlockSpec((BN, MEM), lambda i: (i, 0)),
            pl.BlockSpec((BN, 1), lambda i: (i, 0)),
            pl.BlockSpec((MEM, D), lambda i: (0, 0)),
            pl.BlockSpec((MEM, D), lambda i: (0, 0)),
            pl.BlockSpec((1, D), lambda i: (0, 0)),
        ],
        out_specs=[pl.BlockSpec((BN, D), lambda i: (i, 0)),
                   pl.BlockSpec((BN, D), lambda i: (i, 0))],
        out_shape=[jax.ShapeDtypeStruct((N, D), f32),
                   jax.ShapeDtypeStruct((N, D), f32)],
    )(memory, lu, w1a, w1b, wt)

    BE = 8000
    c = pl.pallas_call(
        _edge_const_body,
        grid=(E // BE,),
        in_specs=[
            pl.BlockSpec((BE, EDGE_F), lambda i: (i, 0)),
            pl.BlockSpec((BE, 1), lambda i: (i, 0)),
            pl.BlockSpec((EDGE_F, D), lambda i: (0, 0)),
            pl.BlockSpec((1, D), lambda i: (0, 0)),
            pl.BlockSpec((1, D), lambda i: (0, 0)),
        ],
        out_specs=pl.BlockSpec((BE, D), lambda i: (i, 0)),
        out_shape=jax.ShapeDtypeStruct((E, D), f32),
    )(edge_features, ts, w1d, wt, b1p)

    zrow = jnp.zeros((ZR, D), f32)
    z16 = jnp.zeros((ZR, CW), f32)
    ones = jnp.ones((K, CW), f32)

    mesh = plsc.VectorSubcoreMesh(core_axis_name="c", subcore_axis_name="s")
    hs, cs = pl.kernel(
        _sc_body,
        out_type=[jax.ShapeDtypeStruct((NC, NP, D), f32),
                  jax.ShapeDtypeStruct((NC, NP, CW), f32)],
        mesh=mesh,
        compiler_params=pltpu.CompilerParams(use_tc_tiling_on_sc=False),
        scratch_types=[
            pltpu.VMEM((K,), jnp.int32),
            pltpu.VMEM((K,), jnp.int32),
            pltpu.VMEM((K, D), f32),
            pltpu.VMEM((K, D), f32),
            pltpu.VMEM((K, D), f32),
            pltpu.VMEM((K, CW), f32),
            pltpu.VMEM_SHARED((NP, D), f32),
            pltpu.VMEM_SHARED((NP, CW), f32),
            pltpu.SemaphoreType.DMA,
            pltpu.SemaphoreType.DMA,
        ],
    )(g1, g2, c, sources, destinations, zrow, z16, ones)

    BF = 2000
    out = pl.pallas_call(
        _finish_body,
        grid=(N // BF,),
        in_specs=[
            pl.BlockSpec((NC, BF, D), lambda i: (0, i, 0)),
            pl.BlockSpec((NC, BF, CW), lambda i: (0, i, 0)),
            pl.BlockSpec((BF, MEM), lambda i: (i, 0)),
            pl.BlockSpec((D, MEM), lambda i: (0, 0)),
            pl.BlockSpec((1, MEM), lambda i: (0, 0)),
            pl.BlockSpec((MEM, 3 * MEM), lambda i: (0, 0)),
            pl.BlockSpec((MEM, 3 * MEM), lambda i: (0, 0)),
            pl.BlockSpec((1, 3 * MEM), lambda i: (0, 0)),
            pl.BlockSpec((1, 3 * MEM), lambda i: (0, 0)),
        ],
        out_specs=pl.BlockSpec((BF, MEM), lambda i: (i, 0)),
        out_shape=jax.ShapeDtypeStruct((N, MEM), f32),
    )(hs, cs, memory, w2p, b2[None, :], W_ih, W_hh, b_ih[None, :], b_hh[None, :])
    return out


# speedup vs baseline: 4.2782x; 1.1326x over previous
import jax
import jax.numpy as jnp
from jax import lax
from jax.experimental import pallas as pl
from jax.experimental.pallas import tpu as pltpu
from jax.experimental.pallas import tpu_sc as plsc

N = 10000
E = 320000
MEM = 128
EDGE_F = 16
RAW = 2 * MEM + 1 + EDGE_F
HID = RAW // 2
D = 144

NC = 2
NS = 16
NW = NC * NS
K = 24
EPT = 10080
E2 = NW * EPT
NCH = EPT // K
NP = 10016
ZR = NP // NS
CW = 8


def _node_proj_body(mem_ref, lu_ref, w1a_ref, w1b_ref, wt_ref, g1_ref, g2_ref):
    m = mem_ref[...]
    g1_ref[...] = (jnp.dot(m, w1a_ref[...], preferred_element_type=jnp.float32)
                   - lu_ref[...] * wt_ref[...])
    g2_ref[...] = jnp.dot(m, w1b_ref[...], preferred_element_type=jnp.float32)


def _edge_const_body(ef_ref, ts_ref, w1d_ref, wt_ref, b1_ref, c_ref):
    c_ref[...] = (jnp.dot(ef_ref[...], w1d_ref[...], preferred_element_type=jnp.float32)
                  + ts_ref[...] * wt_ref[...] + b1_ref[...])


def _sc_body(g1_hbm, g2_hbm, c_hbm, src_hbm, dst_hbm, zrow_hbm, z16_hbm, ones_hbm,
             h_out, cnt_out,
             src0, src1, src2, src3, dst0, dst1, dst2, dst3,
             g1r0, g1r1, g2r0, g2r1, cr0, cr1, hb0, hb1, ones_v, h_sh, c_sh,
             isem0, isem1, isem2, isem3, gsem0, gsem1, ssem0, ssem1):
    cid = lax.axis_index("c")
    sid = lax.axis_index("s")
    pltpu.sync_copy(zrow_hbm, h_sh.at[pl.ds(sid * ZR, ZR)])
    pltpu.sync_copy(z16_hbm, c_sh.at[pl.ds(sid * ZR, ZR)])
    pltpu.sync_copy(ones_hbm, ones_v)
    plsc.subcore_barrier()

    base = (cid * NS + sid) * EPT
    srcs = [src0, src1, src2, src3]
    dsts = [dst0, dst1, dst2, dst3]
    isems = [isem0, isem1, isem2, isem3]
    g1rs = [g1r0, g1r1]
    g2rs = [g2r0, g2r1]
    crs = [cr0, cr1]
    hbs = [hb0, hb1]
    gsems = [gsem0, gsem1]
    ssems = [ssem0, ssem1]

    def off_of(ch):
        return pl.multiple_of(base + ch * K, 8)

    def fetch_idx(ch, q):
        off = off_of(ch)
        pltpu.make_async_copy(src_hbm.at[pl.ds(off, K)], srcs[q], isems[q]).start()
        pltpu.make_async_copy(dst_hbm.at[pl.ds(off, K)], dsts[q], isems[q]).start()

    def wait_idx(q):
        pltpu.make_async_copy(src_hbm.at[pl.ds(0, K)], srcs[q], isems[q]).wait()
        pltpu.make_async_copy(dst_hbm.at[pl.ds(0, K)], dsts[q], isems[q]).wait()

    def issue_gathers(ch, b, q):
        pltpu.make_async_copy(g1_hbm.at[srcs[q]], g1rs[b], gsems[b]).start()
        pltpu.make_async_copy(g2_hbm.at[dsts[q]], g2rs[b], gsems[b]).start()
        pltpu.make_async_copy(c_hbm.at[pl.ds(off_of(ch), K)], crs[b], gsems[b]).start()

    def wait_gathers(b):
        pltpu.make_async_copy(g1_hbm.at[srcs[0]], g1rs[b], gsems[b]).wait()
        pltpu.make_async_copy(g2_hbm.at[dsts[0]], g2rs[b], gsems[b]).wait()
        pltpu.make_async_copy(c_hbm.at[pl.ds(0, K)], crs[b], gsems[b]).wait()

    def issue_scatter(b, q):
        pltpu.make_async_copy(hbs[b], h_sh.at[srcs[q]], ssems[b]).start(add=True)
        pltpu.make_async_copy(ones_v, c_sh.at[srcs[q]], ssems[b]).start(add=True)

    def wait_scatter(b):
        pltpu.make_async_copy(hbs[b], h_sh.at[srcs[0]], ssems[b]).wait()
        pltpu.make_async_copy(ones_v, c_sh.at[srcs[0]], ssems[b]).wait()

    def compute(b):
        g1r, g2r, cr, hb = g1rs[b], g2rs[b], crs[b], hbs[b]

        @pl.loop(0, K)
        def _row(r):
            for j in range(D // 16):
                s = pl.ds(j * 16, 16)
                hb[r, s] = jnp.maximum(g1r[r, s] + g2r[r, s] + cr[r, s], 0.0)

    def process(ch, b, q, qn, qf, cond_next, cond_scat, cond_fetch):
        @pl.when(cond_next)
        def _():
            wait_idx(qn)
            issue_gathers(ch + 1, 1 - b, qn)
        wait_gathers(b)

        @pl.when(cond_scat)
        def _():
            wait_scatter(b)

        @pl.when(cond_fetch)
        def _():
            fetch_idx(ch + 2, qf)
        compute(b)
        issue_scatter(b, q)

    true_ = jnp.bool_(True)
    fetch_idx(0, 0)
    fetch_idx(1, 1)
    wait_idx(0)
    issue_gathers(0, 0, 0)

    @pl.loop(0, NCH // 4)
    def _grp(g):
        ch0 = g * 4
        not_last = g < NCH // 4 - 1
        not_first = g > 0
        process(ch0 + 0, 0, 0, 1, 2, true_, not_first, true_)
        process(ch0 + 1, 1, 1, 2, 3, true_, not_first, true_)
        process(ch0 + 2, 0, 2, 3, 0, true_, true_, not_last)
        process(ch0 + 3, 1, 3, 0, 1, not_last, true_, not_last)

    wait_scatter(0)
    wait_scatter(1)

    plsc.subcore_barrier()
    pltpu.sync_copy(h_sh.at[pl.ds(sid * ZR, ZR)], h_out.at[cid, pl.ds(sid * ZR, ZR)])
    pltpu.sync_copy(c_sh.at[pl.ds(sid * ZR, ZR)], cnt_out.at[cid, pl.ds(sid * ZR, ZR)])


def _finish_body(hs_ref, cs_ref, mem_ref, w2_ref, b2_ref, wih_ref, whh_ref,
                 bih_ref, bhh_ref, out_ref):
    hsum = hs_ref[0] + hs_ref[1]
    cnt = cs_ref[0, :, 0:1] + cs_ref[1, :, 0:1]
    mean = hsum / jnp.maximum(cnt, 1.0)
    agg = jnp.dot(mean, w2_ref[...], preferred_element_type=jnp.float32) + b2_ref[...]
    m = mem_ref[...]
    gi = jnp.dot(agg, wih_ref[...], preferred_element_type=jnp.float32) + bih_ref[...]
    gh = jnp.dot(m, whh_ref[...], preferred_element_type=jnp.float32) + bhh_ref[...]
    r = jax.nn.sigmoid(gi[:, :MEM] + gh[:, :MEM])
    z = jax.nn.sigmoid(gi[:, MEM:2 * MEM] + gh[:, MEM:2 * MEM])
    n = jnp.tanh(gi[:, 2 * MEM:] + r * gh[:, 2 * MEM:])
    new = (1.0 - z) * n + z * m
    out_ref[...] = jnp.where(cnt > 0.0, new, m)


def kernel(sources, destinations, timestamps, edge_features, memory, last_update,
           W1, b1, W2, b2, W_ih, W_hh, b_ih, b_hh):
    f32 = jnp.float32
    pad = lambda w: jnp.pad(w, ((0, 0), (0, D - HID)))
    w1a = pad(W1[:MEM])
    w1b = pad(W1[MEM:2 * MEM])
    wt = pad(W1[2 * MEM:2 * MEM + 1])
    w1d = pad(W1[2 * MEM + 1:])
    b1p = pad(b1[None, :])
    w2p = jnp.pad(W2, ((0, D - HID), (0, 0)))
    lu = last_update[:, None]
    ts = timestamps[:, None]
    src_p = jnp.pad(sources, (0, E2 - E), constant_values=N)
    dst_p = jnp.pad(destinations, (0, E2 - E), constant_values=N)

    BN = 2000
    g1, g2 = pl.pallas_call(
        _node_proj_body,
        grid=(N // BN,),
        in_specs=[
            pl.BlockSpec((BN, MEM), lambda i: (i, 0)),
            pl.BlockSpec((BN, 1), lambda i: (i, 0)),
            pl.BlockSpec((MEM, D), lambda i: (0, 0)),
            pl.BlockSpec((MEM, D), lambda i: (0, 0)),
            pl.BlockSpec((1, D), lambda i: (0, 0)),
        ],
        out_specs=[pl.BlockSpec((BN, D), lambda i: (i, 0)),
                   pl.BlockSpec((BN, D), lambda i: (i, 0))],
        out_shape=[jax.ShapeDtypeStruct((NP, D), f32),
                   jax.ShapeDtypeStruct((NP, D), f32)],
    )(memory, lu, w1a, w1b, wt)

    BE = 8000
    c = pl.pallas_call(
        _edge_const_body,
        grid=(E // BE,),
        in_specs=[
            pl.BlockSpec((BE, EDGE_F), lambda i: (i, 0)),
            pl.BlockSpec((BE, 1), lambda i: (i, 0)),
            pl.BlockSpec((EDGE_F, D), lambda i: (0, 0)),
            pl.BlockSpec((1, D), lambda i: (0, 0)),
            pl.BlockSpec((1, D), lambda i: (0, 0)),
        ],
        out_specs=pl.BlockSpec((BE, D), lambda i: (i, 0)),
        out_shape=jax.ShapeDtypeStruct((E2, D), f32),
    )(edge_features, ts, w1d, wt, b1p)

    zrow = jnp.zeros((ZR, D), f32)
    z16 = jnp.zeros((ZR, CW), f32)
    ones = jnp.ones((K, CW), f32)

    mesh = plsc.VectorSubcoreMesh(core_axis_name="c", subcore_axis_name="s")
    hs, cs = pl.kernel(
        _sc_body,
        out_type=[jax.ShapeDtypeStruct((NC, NP, D), f32),
                  jax.ShapeDtypeStruct((NC, NP, CW), f32)],
        mesh=mesh,
        compiler_params=pltpu.CompilerParams(use_tc_tiling_on_sc=False),
        scratch_types=[
            pltpu.VMEM((K,), jnp.int32), pltpu.VMEM((K,), jnp.int32),
            pltpu.VMEM((K,), jnp.int32), pltpu.VMEM((K,), jnp.int32),
            pltpu.VMEM((K,), jnp.int32), pltpu.VMEM((K,), jnp.int32),
            pltpu.VMEM((K,), jnp.int32), pltpu.VMEM((K,), jnp.int32),
            pltpu.VMEM((K, D), f32), pltpu.VMEM((K, D), f32),
            pltpu.VMEM((K, D), f32), pltpu.VMEM((K, D), f32),
            pltpu.VMEM((K, D), f32), pltpu.VMEM((K, D), f32),
            pltpu.VMEM((K, D), f32), pltpu.VMEM((K, D), f32),
            pltpu.VMEM((K, CW), f32),
            pltpu.VMEM_SHARED((NP, D), f32),
            pltpu.VMEM_SHARED((NP, CW), f32),
            pltpu.SemaphoreType.DMA, pltpu.SemaphoreType.DMA,
            pltpu.SemaphoreType.DMA, pltpu.SemaphoreType.DMA,
            pltpu.SemaphoreType.DMA, pltpu.SemaphoreType.DMA,
            pltpu.SemaphoreType.DMA, pltpu.SemaphoreType.DMA,
        ],
    )(g1, g2, c, src_p, dst_p, zrow, z16, ones)

    BF = 2000
    out = pl.pallas_call(
        _finish_body,
        grid=(N // BF,),
        in_specs=[
            pl.BlockSpec((NC, BF, D), lambda i: (0, i, 0)),
            pl.BlockSpec((NC, BF, CW), lambda i: (0, i, 0)),
            pl.BlockSpec((BF, MEM), lambda i: (i, 0)),
            pl.BlockSpec((D, MEM), lambda i: (0, 0)),
            pl.BlockSpec((1, MEM), lambda i: (0, 0)),
            pl.BlockSpec((MEM, 3 * MEM), lambda i: (0, 0)),
            pl.BlockSpec((MEM, 3 * MEM), lambda i: (0, 0)),
            pl.BlockSpec((1, 3 * MEM), lambda i: (0, 0)),
            pl.BlockSpec((1, 3 * MEM), lambda i: (0, 0)),
        ],
        out_specs=pl.BlockSpec((BF, MEM), lambda i: (i, 0)),
        out_shape=jax.ShapeDtypeStruct((N, MEM), f32),
    )(hs, cs, memory, w2p, b2[None, :], W_ih, W_hh, b_ih[None, :], b_hh[None, :])
    return out
